# Initial kernel scaffold; baseline (speedup 1.0000x reference)
import jax
import jax.numpy as jnp
from jax.experimental import pallas as pl


def kernel(x, edge_index, W1, b1, g1, be1, W2, b2, g2, be2, W3, b3, g3, be3, fw1, fb1, fw2, fb2, fw3, fb3, fw4, fb4):
    raise NotImplementedError("write your pallas kernel here")


# speedup vs baseline: 30.0986x; 30.0986x over previous
import functools

import jax
import jax.numpy as jnp
from jax import lax
from jax.experimental import pallas as pl
from jax.experimental.pallas import tpu as pltpu
from jax.experimental.pallas import tpu_sc as plsc

N = 10000
E = 320000
H = 16
D = 128

NC = 2
NS = 16
NW = NC * NS
EPT = 10240
EP = NW * EPT
GC = 1024
NG = EPT // GC
SB = GC // 128
ACC_ROWS = 10112
RPT = ACC_ROWS // NS

_MESH = plsc.VectorSubcoreMesh(core_axis_name="c", subcore_axis_name="s")
_SC_PARAMS = pltpu.CompilerParams(use_tc_tiling_on_sc=False)


def _prop_body(hp_hbm, src_hbm, dst_hbm, zeros_hbm, out_hbm,
               src_v, dst_v, rows_v, zrow_v, sem, acc_sh):
    c = lax.axis_index("c")
    s = lax.axis_index("s")
    tid = c * NS + s
    pltpu.sync_copy(src_hbm.at[pl.ds(tid * EPT, EPT)], src_v)
    pltpu.sync_copy(dst_hbm.at[tid], dst_v)
    pltpu.sync_copy(zeros_hbm, zrow_v)
    pltpu.sync_copy(zrow_v, acc_sh.at[pl.ds(s * RPT, RPT)])
    plsc.subcore_barrier()

    def chunk(g, carry):
        pltpu.async_copy(
            hp_hbm.at[src_v.at[pl.ds(g * GC, GC)]], rows_v, sem).wait()

        def sub(j, carry2):
            pltpu.sync_copy(rows_v.at[pl.ds(j * 128, 128)],
                            acc_sh.at[dst_v.at[g * SB + j]], add=True)
            return carry2

        lax.fori_loop(0, SB, sub, 0)
        return carry

    lax.fori_loop(0, NG, chunk, 0)
    plsc.subcore_barrier()
    pltpu.sync_copy(acc_sh.at[pl.ds(s * RPT, RPT)],
                    out_hbm.at[c].at[pl.ds(s * RPT, RPT)])


_prop_call = pl.kernel(
    _prop_body,
    out_type=jax.ShapeDtypeStruct((NC, ACC_ROWS, H), jnp.float32),
    mesh=_MESH,
    scratch_types=[
        pltpu.VMEM((EPT,), jnp.int32),
        pltpu.VMEM((NG * SB, 128), jnp.int32),
        pltpu.VMEM((GC, H), jnp.float32),
        pltpu.VMEM((RPT, H), jnp.float32),
        pltpu.SemaphoreType.DMA,
        pltpu.VMEM_SHARED((ACC_ROWS, H), jnp.float32),
    ],
    compiler_params=_SC_PARAMS,
)


def _deg_body(dst_hbm, ones_hbm, zeros_hbm, out_hbm,
              dst_v, ones_v, zrow_v, acc_sh):
    c = lax.axis_index("c")
    s = lax.axis_index("s")
    tid = c * NS + s
    pltpu.sync_copy(dst_hbm.at[tid], dst_v)
    pltpu.sync_copy(ones_hbm, ones_v)
    pltpu.sync_copy(zeros_hbm, zrow_v)
    pltpu.sync_copy(zrow_v, acc_sh.at[pl.ds(s * RPT, RPT)])
    plsc.subcore_barrier()

    def chunk(g, carry):
        pltpu.sync_copy(ones_v, acc_sh.at[dst_v.at[g]], add=True)
        return carry

    lax.fori_loop(0, NG * SB, chunk, 0)
    plsc.subcore_barrier()
    pltpu.sync_copy(acc_sh.at[pl.ds(s * RPT, RPT)],
                    out_hbm.at[c].at[pl.ds(s * RPT, RPT)])


_deg_call = pl.kernel(
    _deg_body,
    out_type=jax.ShapeDtypeStruct((NC, ACC_ROWS, H), jnp.float32),
    mesh=_MESH,
    scratch_types=[
        pltpu.VMEM((NG * SB, 128), jnp.int32),
        pltpu.VMEM((128, H), jnp.float32),
        pltpu.VMEM((RPT, H), jnp.float32),
        pltpu.VMEM_SHARED((ACC_ROWS, H), jnp.float32),
    ],
    compiler_params=_SC_PARAMS,
)


def _tc_pre_body(degacc_ref, x_ref, w1_ref, dinv_ref, hp1_ref):
    deg = degacc_ref[0, :N, 0:1] + degacc_ref[1, :N, 0:1] + 1.0
    dinv = lax.rsqrt(deg)
    t1 = jnp.dot(x_ref[...], w1_ref[...], preferred_element_type=jnp.float32)
    dinv_ref[...] = dinv
    hp1_ref[...] = t1 * dinv


def _bn_gelu(p, g, be):
    m = jnp.mean(p, axis=0, keepdims=True)
    v = jnp.mean((p - m) ** 2, axis=0, keepdims=True)
    return jax.nn.gelu((p - m) * lax.rsqrt(v + 1e-5) * g + be)


def _tc_mid_body(acc_ref, hp_ref, dinv_ref, b_ref, g_ref, be_ref, wn_ref,
                 hpn_ref):
    dinv = dinv_ref[...]
    p = (acc_ref[0, :N, :] + acc_ref[1, :N, :] + hp_ref[...]) * dinv
    u = _bn_gelu(p + b_ref[...], g_ref[...], be_ref[...])
    hpn_ref[...] = jnp.dot(
        u, wn_ref[...], preferred_element_type=jnp.float32) * dinv


def _tc_post_body(acc_ref, hp_ref, dinv_ref, x_ref, w3_ref, b3_ref, g3_ref,
                  be3_ref, fw1_ref, fb1_ref, fw2_ref, fb2_ref, fw3_ref,
                  fb3_ref, fw4_ref, fb4_ref, out_ref):
    p3 = (acc_ref[0, :N, :] + acc_ref[1, :N, :] + hp_ref[...]) * dinv_ref[...]
    t3 = jnp.dot(p3, w3_ref[...], preferred_element_type=jnp.float32)
    u3 = _bn_gelu(t3 + b3_ref[...], g3_ref[...], be3_ref[...])
    v = jnp.mean(x_ref[...] + u3, axis=0, keepdims=True)
    v = jax.nn.gelu(
        jnp.dot(v, fw1_ref[...], preferred_element_type=jnp.float32)
        + fb1_ref[...])
    v = jax.nn.gelu(
        jnp.dot(v, fw2_ref[...], preferred_element_type=jnp.float32)
        + fb2_ref[...])
    v = jax.nn.gelu(
        jnp.dot(v, fw3_ref[...], preferred_element_type=jnp.float32)
        + fb3_ref[...])
    out_ref[...] = (jnp.sum(v * fw4_ref[...], axis=1, keepdims=True)
                    + fb4_ref[...])


def _tc_call(body, out_shapes):
    return pl.pallas_call(body, out_shape=out_shapes)


_pre_call = _tc_call(_tc_pre_body, (
    jax.ShapeDtypeStruct((N, 1), jnp.float32),
    jax.ShapeDtypeStruct((N, H), jnp.float32),
))
_mid_call = _tc_call(_tc_mid_body, jax.ShapeDtypeStruct((N, H), jnp.float32))
_post_call = _tc_call(_tc_post_body, jax.ShapeDtypeStruct((1, 1), jnp.float32))


def kernel(x, edge_index, W1, b1, g1, be1, W2, b2, g2, be2, W3, b3, g3, be3,
           fw1, fb1, fw2, fb2, fw3, fb3, fw4, fb4):
    ei = edge_index.astype(jnp.int32)
    src = jnp.concatenate([ei[0], jnp.zeros((EP - E,), jnp.int32)])
    dst = jnp.concatenate([ei[1], jnp.full((EP - E,), N, jnp.int32)])
    dst = dst.reshape(NW, NG * SB, 128)

    zeros_rows = jnp.zeros((RPT, H), jnp.float32)
    ones_rows = jnp.ones((128, H), jnp.float32)

    degacc = _deg_call(dst, ones_rows, zeros_rows)
    dinv, hp1 = _pre_call(degacc, x, W1)

    acc1 = _prop_call(hp1, src, dst, zeros_rows)
    hp2 = _mid_call(acc1, hp1, dinv, b1.reshape(1, H), g1.reshape(1, H),
                    be1.reshape(1, H), W2)

    acc2 = _prop_call(hp2, src, dst, zeros_rows)
    hp3 = _mid_call(acc2, hp2, dinv, b2.reshape(1, H), g2.reshape(1, H),
                    be2.reshape(1, H), jnp.eye(H, dtype=jnp.float32))

    acc3 = _prop_call(hp3, src, dst, zeros_rows)
    out = _post_call(acc3, hp3, dinv, x, W3, b3.reshape(1, D),
                     g3.reshape(1, D), be3.reshape(1, D), fw1,
                     fb1.reshape(1, D), fw2, fb2.reshape(1, 64), fw3,
                     fb3.reshape(1, 32), fw4.reshape(1, 32),
                     fb4.reshape(1, 1))
    return out.reshape(1)


# speedup vs baseline: 32.1457x; 1.0680x over previous
import functools

import jax
import jax.numpy as jnp
from jax import lax
from jax.experimental import pallas as pl
from jax.experimental.pallas import tpu as pltpu
from jax.experimental.pallas import tpu_sc as plsc

N = 10000
E = 320000
H = 16
D = 128

NC = 2
NS = 16
NW = NC * NS
EPT = 10240
EP = NW * EPT
GC = 2048
NG = EPT // GC
ACC_ROWS = 10112
RPT = ACC_ROWS // NS

_MESH = plsc.VectorSubcoreMesh(core_axis_name="c", subcore_axis_name="s")
_SC_PARAMS = pltpu.CompilerParams(use_tc_tiling_on_sc=False)


def _prop_body(hp_hbm, src_hbm, dst_hbm, zeros_hbm, out_hbm,
               src_v, dst_v, rows_v, zrow_v, sem, acc_sh):
    c = lax.axis_index("c")
    s = lax.axis_index("s")
    tid = c * NS + s
    pltpu.sync_copy(src_hbm.at[pl.ds(tid * EPT, EPT)], src_v)
    pltpu.sync_copy(dst_hbm.at[tid], dst_v)
    pltpu.sync_copy(zeros_hbm, zrow_v)
    pltpu.sync_copy(zrow_v, acc_sh.at[pl.ds(s * RPT, RPT)])
    plsc.subcore_barrier()

    def gather(g, buf):
        pltpu.async_copy(
            hp_hbm.at[src_v.at[pl.ds(g * GC, GC)]], rows_v.at[buf], sem)

    gather(0, 0)

    def chunk(g, carry):
        pltpu.make_async_copy(
            hp_hbm.at[src_v.at[pl.ds(0, GC)]], rows_v.at[0], sem).wait()

        @pl.when(g + 1 < NG)
        def _():
            gather(g + 1, (g + 1) % 2)

        pltpu.sync_copy(rows_v.at[g % 2], acc_sh.at[dst_v.at[g]], add=True)
        return carry

    lax.fori_loop(0, NG, chunk, 0)
    plsc.subcore_barrier()
    pltpu.sync_copy(acc_sh.at[pl.ds(s * RPT, RPT)],
                    out_hbm.at[c].at[pl.ds(s * RPT, RPT)])


_prop_call = pl.kernel(
    _prop_body,
    out_type=jax.ShapeDtypeStruct((NC, ACC_ROWS, H), jnp.float32),
    mesh=_MESH,
    scratch_types=[
        pltpu.VMEM((EPT,), jnp.int32),
        pltpu.VMEM((NG, GC), jnp.int32),
        pltpu.VMEM((2, GC, H), jnp.float32),
        pltpu.VMEM((RPT, H), jnp.float32),
        pltpu.SemaphoreType.DMA,
        pltpu.VMEM_SHARED((ACC_ROWS, H), jnp.float32),
    ],
    compiler_params=_SC_PARAMS,
)


def _deg_body(dst_hbm, ones_hbm, zeros_hbm, out_hbm,
              dst_v, ones_v, zrow_v, acc_sh):
    c = lax.axis_index("c")
    s = lax.axis_index("s")
    tid = c * NS + s
    pltpu.sync_copy(dst_hbm.at[tid], dst_v)
    pltpu.sync_copy(ones_hbm, ones_v)
    pltpu.sync_copy(zeros_hbm, zrow_v)
    pltpu.sync_copy(zrow_v, acc_sh.at[pl.ds(s * RPT, RPT)])
    plsc.subcore_barrier()

    def chunk(g, carry):
        pltpu.sync_copy(ones_v, acc_sh.at[dst_v.at[g]], add=True)
        return carry

    lax.fori_loop(0, NG, chunk, 0)
    plsc.subcore_barrier()
    pltpu.sync_copy(acc_sh.at[pl.ds(s * RPT, RPT)],
                    out_hbm.at[c].at[pl.ds(s * RPT, RPT)])


_deg_call = pl.kernel(
    _deg_body,
    out_type=jax.ShapeDtypeStruct((NC, ACC_ROWS, H), jnp.float32),
    mesh=_MESH,
    scratch_types=[
        pltpu.VMEM((NG, GC), jnp.int32),
        pltpu.VMEM((GC, H), jnp.float32),
        pltpu.VMEM((RPT, H), jnp.float32),
        pltpu.VMEM_SHARED((ACC_ROWS, H), jnp.float32),
    ],
    compiler_params=_SC_PARAMS,
)


def _tc_pre_body(degacc_ref, x_ref, w1_ref, dinv_ref, hp1_ref):
    deg = degacc_ref[0, :N, 0:1] + degacc_ref[1, :N, 0:1] + 1.0
    dinv = lax.rsqrt(deg)
    t1 = jnp.dot(x_ref[...], w1_ref[...], preferred_element_type=jnp.float32)
    dinv_ref[...] = dinv
    hp1_ref[...] = t1 * dinv


def _bn_gelu(p, g, be):
    m = jnp.mean(p, axis=0, keepdims=True)
    v = jnp.mean((p - m) ** 2, axis=0, keepdims=True)
    return jax.nn.gelu((p - m) * lax.rsqrt(v + 1e-5) * g + be)


def _tc_mid_body(acc_ref, hp_ref, dinv_ref, b_ref, g_ref, be_ref, wn_ref,
                 hpn_ref):
    dinv = dinv_ref[...]
    p = (acc_ref[0, :N, :] + acc_ref[1, :N, :] + hp_ref[...]) * dinv
    u = _bn_gelu(p + b_ref[...], g_ref[...], be_ref[...])
    hpn_ref[...] = jnp.dot(
        u, wn_ref[...], preferred_element_type=jnp.float32) * dinv


def _tc_post_body(acc_ref, hp_ref, dinv_ref, x_ref, w3_ref, b3_ref, g3_ref,
                  be3_ref, fw1_ref, fb1_ref, fw2_ref, fb2_ref, fw3_ref,
                  fb3_ref, fw4_ref, fb4_ref, out_ref):
    p3 = (acc_ref[0, :N, :] + acc_ref[1, :N, :] + hp_ref[...]) * dinv_ref[...]
    t3 = jnp.dot(p3, w3_ref[...], preferred_element_type=jnp.float32)
    u3 = _bn_gelu(t3 + b3_ref[...], g3_ref[...], be3_ref[...])
    v = jnp.mean(x_ref[...] + u3, axis=0, keepdims=True)
    v = jax.nn.gelu(
        jnp.dot(v, fw1_ref[...], preferred_element_type=jnp.float32)
        + fb1_ref[...])
    v = jax.nn.gelu(
        jnp.dot(v, fw2_ref[...], preferred_element_type=jnp.float32)
        + fb2_ref[...])
    v = jax.nn.gelu(
        jnp.dot(v, fw3_ref[...], preferred_element_type=jnp.float32)
        + fb3_ref[...])
    out_ref[...] = (jnp.sum(v * fw4_ref[...], axis=1, keepdims=True)
                    + fb4_ref[...])


def _tc_call(body, out_shapes):
    return pl.pallas_call(body, out_shape=out_shapes)


_pre_call = _tc_call(_tc_pre_body, (
    jax.ShapeDtypeStruct((N, 1), jnp.float32),
    jax.ShapeDtypeStruct((N, H), jnp.float32),
))
_mid_call = _tc_call(_tc_mid_body, jax.ShapeDtypeStruct((N, H), jnp.float32))
_post_call = _tc_call(_tc_post_body, jax.ShapeDtypeStruct((1, 1), jnp.float32))


def kernel(x, edge_index, W1, b1, g1, be1, W2, b2, g2, be2, W3, b3, g3, be3,
           fw1, fb1, fw2, fb2, fw3, fb3, fw4, fb4):
    ei = edge_index.astype(jnp.int32)
    src = jnp.concatenate([ei[0], jnp.zeros((EP - E,), jnp.int32)])
    dst = jnp.concatenate([ei[1], jnp.full((EP - E,), N, jnp.int32)])
    dst = dst.reshape(NW, NG, GC)

    zeros_rows = jnp.zeros((RPT, H), jnp.float32)
    ones_rows = jnp.ones((GC, H), jnp.float32)

    degacc = _deg_call(dst, ones_rows, zeros_rows)
    dinv, hp1 = _pre_call(degacc, x, W1)

    acc1 = _prop_call(hp1, src, dst, zeros_rows)
    hp2 = _mid_call(acc1, hp1, dinv, b1.reshape(1, H), g1.reshape(1, H),
                    be1.reshape(1, H), W2)

    acc2 = _prop_call(hp2, src, dst, zeros_rows)
    hp3 = _mid_call(acc2, hp2, dinv, b2.reshape(1, H), g2.reshape(1, H),
                    be2.reshape(1, H), jnp.eye(H, dtype=jnp.float32))

    acc3 = _prop_call(hp3, src, dst, zeros_rows)
    out = _post_call(acc3, hp3, dinv, x, W3, b3.reshape(1, D),
                     g3.reshape(1, D), be3.reshape(1, D), fw1,
                     fb1.reshape(1, D), fw2, fb2.reshape(1, 64), fw3,
                     fb3.reshape(1, 32), fw4.reshape(1, 32),
                     fb4.reshape(1, 1))
    return out.reshape(1)


# speedup vs baseline: 53.5376x; 1.6655x over previous
import jax
import jax.numpy as jnp
from jax import lax
from jax.experimental import pallas as pl
from jax.experimental.pallas import tpu as pltpu
from jax.experimental.pallas import tpu_sc as plsc

N = 10000
E = 320000
H = 16
D = 128

NC = 2
NS = 16
NW = NC * NS
EPT = E // NW
GC = 2000
NG = EPT // GC
TBL_ROWS = 10112
RPT = TBL_ROWS // NS

_MESH = plsc.VectorSubcoreMesh(core_axis_name="c", subcore_axis_name="s")
_SC_PARAMS = pltpu.CompilerParams(use_tc_tiling_on_sc=False)


def _prop_body(hp_hbm, ei_hbm, zeros_hbm, out_hbm,
               srcix_v, dstix_v, rows_v, isem, gsem, acc_sh, hp_sh):
    c = lax.axis_index("c")
    s = lax.axis_index("s")
    base = (c * NS + s) * EPT
    pltpu.sync_copy(zeros_hbm, acc_sh.at[pl.ds(s * RPT, RPT)])
    pltpu.sync_copy(hp_hbm.at[pl.ds(s * RPT, RPT)],
                    hp_sh.at[pl.ds(s * RPT, RPT)])

    def load_idx(g, buf):
        pltpu.async_copy(
            ei_hbm.at[0].at[pl.ds(base + g * GC, GC)], srcix_v.at[buf], isem)
        pltpu.async_copy(
            ei_hbm.at[1].at[pl.ds(base + g * GC, GC)], dstix_v.at[buf], isem)

    def wait_idx():
        pltpu.make_async_copy(
            ei_hbm.at[0].at[pl.ds(0, GC)], srcix_v.at[0], isem).wait()
        pltpu.make_async_copy(
            ei_hbm.at[1].at[pl.ds(0, GC)], dstix_v.at[0], isem).wait()

    load_idx(0, 0)
    wait_idx()
    plsc.subcore_barrier()
    pltpu.async_copy(hp_sh.at[srcix_v.at[0]], rows_v.at[0], gsem)
    load_idx(1, 1)

    def chunk(g, carry):
        pltpu.make_async_copy(
            hp_sh.at[srcix_v.at[0]], rows_v.at[0], gsem).wait()

        @pl.when(g + 1 < NG)
        def _():
            wait_idx()
            pltpu.async_copy(
                hp_sh.at[srcix_v.at[(g + 1) % 2]], rows_v.at[(g + 1) % 2],
                gsem)

        pltpu.sync_copy(rows_v.at[g % 2], acc_sh.at[dstix_v.at[g % 2]],
                        add=True)

        @pl.when(g + 2 < NG)
        def _():
            load_idx(g + 2, g % 2)

        return carry

    lax.fori_loop(0, NG, chunk, 0)
    plsc.subcore_barrier()
    pltpu.sync_copy(acc_sh.at[pl.ds(s * RPT, RPT)],
                    out_hbm.at[c].at[pl.ds(s * RPT, RPT)])


_prop_call = pl.kernel(
    _prop_body,
    out_type=jax.ShapeDtypeStruct((NC, TBL_ROWS, H), jnp.float32),
    mesh=_MESH,
    scratch_types=[
        pltpu.VMEM((2, GC), jnp.int32),
        pltpu.VMEM((2, GC), jnp.int32),
        pltpu.VMEM((2, GC, H), jnp.float32),
        pltpu.SemaphoreType.DMA,
        pltpu.SemaphoreType.DMA,
        pltpu.VMEM_SHARED((TBL_ROWS, H), jnp.float32),
        pltpu.VMEM_SHARED((TBL_ROWS, H), jnp.float32),
    ],
    compiler_params=_SC_PARAMS,
)


def _deg_body(ei_hbm, ones_hbm, zeros_hbm, out_hbm,
              dstix_v, ones_v, isem, acc_sh):
    c = lax.axis_index("c")
    s = lax.axis_index("s")
    base = (c * NS + s) * EPT
    pltpu.sync_copy(zeros_hbm, acc_sh.at[pl.ds(s * RPT, RPT)])
    pltpu.sync_copy(ones_hbm, ones_v)
    pltpu.async_copy(ei_hbm.at[1].at[pl.ds(base, GC)], dstix_v.at[0], isem)
    plsc.subcore_barrier()

    def chunk(g, carry):
        pltpu.make_async_copy(
            ei_hbm.at[1].at[pl.ds(0, GC)], dstix_v.at[0], isem).wait()

        @pl.when(g + 1 < NG)
        def _():
            pltpu.async_copy(
                ei_hbm.at[1].at[pl.ds(base + (g + 1) * GC, GC)],
                dstix_v.at[(g + 1) % 2], isem)

        pltpu.sync_copy(ones_v, acc_sh.at[dstix_v.at[g % 2]], add=True)
        return carry

    lax.fori_loop(0, NG, chunk, 0)
    plsc.subcore_barrier()
    pltpu.sync_copy(acc_sh.at[pl.ds(s * RPT, RPT)],
                    out_hbm.at[c].at[pl.ds(s * RPT, RPT)])


_deg_call = pl.kernel(
    _deg_body,
    out_type=jax.ShapeDtypeStruct((NC, TBL_ROWS, H), jnp.float32),
    mesh=_MESH,
    scratch_types=[
        pltpu.VMEM((2, GC), jnp.int32),
        pltpu.VMEM((GC, H), jnp.float32),
        pltpu.SemaphoreType.DMA,
        pltpu.VMEM_SHARED((TBL_ROWS, H), jnp.float32),
    ],
    compiler_params=_SC_PARAMS,
)


def _tc_pre_body(degacc_ref, x_ref, w1_ref, dinv_ref, hp1_ref):
    deg = degacc_ref[0, :N, 0:1] + degacc_ref[1, :N, 0:1] + 1.0
    dinv = lax.rsqrt(deg)
    t1 = jnp.dot(x_ref[...], w1_ref[...], preferred_element_type=jnp.float32)
    dinv_ref[...] = dinv
    hp1_ref[:N, :] = t1 * dinv


def _bn_gelu(p, g, be):
    m = jnp.mean(p, axis=0, keepdims=True)
    v = jnp.mean((p - m) ** 2, axis=0, keepdims=True)
    return jax.nn.gelu((p - m) * lax.rsqrt(v + 1e-5) * g + be)


def _tc_mid_body(acc_ref, hp_ref, dinv_ref, b_ref, g_ref, be_ref, wn_ref,
                 hpn_ref):
    dinv = dinv_ref[...]
    p = (acc_ref[0, :N, :] + acc_ref[1, :N, :] + hp_ref[:N, :]) * dinv
    u = _bn_gelu(p + b_ref[...], g_ref[...], be_ref[...])
    hpn_ref[:N, :] = jnp.dot(
        u, wn_ref[...], preferred_element_type=jnp.float32) * dinv


def _tc_post_body(acc_ref, hp_ref, dinv_ref, x_ref, w3_ref, b3_ref, g3_ref,
                  be3_ref, fw1_ref, fb1_ref, fw2_ref, fb2_ref, fw3_ref,
                  fb3_ref, fw4_ref, fb4_ref, out_ref):
    p3 = (acc_ref[0, :N, :] + acc_ref[1, :N, :] + hp_ref[:N, :]) \
        * dinv_ref[...]
    t3 = jnp.dot(p3, w3_ref[...], preferred_element_type=jnp.float32)
    u3 = _bn_gelu(t3 + b3_ref[...], g3_ref[...], be3_ref[...])
    v = jnp.mean(x_ref[...] + u3, axis=0, keepdims=True)
    v = jax.nn.gelu(
        jnp.dot(v, fw1_ref[...], preferred_element_type=jnp.float32)
        + fb1_ref[...])
    v = jax.nn.gelu(
        jnp.dot(v, fw2_ref[...], preferred_element_type=jnp.float32)
        + fb2_ref[...])
    v = jax.nn.gelu(
        jnp.dot(v, fw3_ref[...], preferred_element_type=jnp.float32)
        + fb3_ref[...])
    out_ref[...] = (jnp.dot(v, fw4_ref[...],
                            preferred_element_type=jnp.float32)
                    + fb4_ref[...])


_pre_call = pl.pallas_call(_tc_pre_body, out_shape=(
    jax.ShapeDtypeStruct((N, 1), jnp.float32),
    jax.ShapeDtypeStruct((TBL_ROWS, H), jnp.float32),
))
_mid_call = pl.pallas_call(
    _tc_mid_body, out_shape=jax.ShapeDtypeStruct((TBL_ROWS, H), jnp.float32))
_post_call = pl.pallas_call(
    _tc_post_body, out_shape=jax.ShapeDtypeStruct((1, 1), jnp.float32))


def kernel(x, edge_index, W1, b1, g1, be1, W2, b2, g2, be2, W3, b3, g3, be3,
           fw1, fb1, fw2, fb2, fw3, fb3, fw4, fb4):
    ei = edge_index.astype(jnp.int32)
    zeros_rows = jnp.zeros((RPT, H), jnp.float32)
    ones_rows = jnp.ones((GC, H), jnp.float32)

    degacc = _deg_call(ei, ones_rows, zeros_rows)
    dinv, hp1 = _pre_call(degacc, x, W1)

    acc1 = _prop_call(hp1, ei, zeros_rows)
    hp2 = _mid_call(acc1, hp1, dinv, b1, g1, be1, W2)

    acc2 = _prop_call(hp2, ei, zeros_rows)
    hp3 = _mid_call(acc2, hp2, dinv, b2, g2, be2, jnp.eye(H, dtype=jnp.float32))

    acc3 = _prop_call(hp3, ei, zeros_rows)
    out = _post_call(acc3, hp3, dinv, x, W3, b3, g3, be3,
                     fw1, fb1, fw2, fb2, fw3, fb3, fw4, fb4)
    return out.reshape(1)


# speedup vs baseline: 83.5989x; 1.5615x over previous
import jax
import jax.numpy as jnp
from jax import lax
from jax.experimental import pallas as pl
from jax.experimental.pallas import tpu as pltpu
from jax.experimental.pallas import tpu_sc as plsc

N = 10000
E = 320000
H = 16
D = 128

NC = 2
NS = 16
NW = NC * NS
EPT = E // NW
GC = 2000
NG = EPT // GC
TBL_ROWS = 10112
RPT = TBL_ROWS // NS
PR = TBL_ROWS * H // 128
PN = N * H // 128

_MESH = plsc.VectorSubcoreMesh(core_axis_name="c", subcore_axis_name="s")
_SC_PARAMS = pltpu.CompilerParams(use_tc_tiling_on_sc=False)


def _prop_body(hp_hbm, ei_hbm, zeros_hbm, out_hbm,
               srcix_v, dstix_v, rows_v, isem, gsem, acc_sh, hp_sh):
    c = lax.axis_index("c")
    s = lax.axis_index("s")
    base = (c * NS + s) * EPT
    pltpu.sync_copy(zeros_hbm, acc_sh.at[pl.ds(s * RPT, RPT)])
    pltpu.sync_copy(hp_hbm.at[pl.ds(s * RPT, RPT)],
                    hp_sh.at[pl.ds(s * RPT, RPT)])

    def load_idx(g, buf):
        pltpu.async_copy(
            ei_hbm.at[0].at[pl.ds(base + g * GC, GC)], srcix_v.at[buf], isem)
        pltpu.async_copy(
            ei_hbm.at[1].at[pl.ds(base + g * GC, GC)], dstix_v.at[buf], isem)

    def wait_idx():
        pltpu.make_async_copy(
            ei_hbm.at[0].at[pl.ds(0, GC)], srcix_v.at[0], isem).wait()
        pltpu.make_async_copy(
            ei_hbm.at[1].at[pl.ds(0, GC)], dstix_v.at[0], isem).wait()

    load_idx(0, 0)
    wait_idx()
    plsc.subcore_barrier()
    pltpu.async_copy(hp_sh.at[srcix_v.at[0]], rows_v.at[0], gsem)
    load_idx(1, 1)

    def chunk(g, carry):
        pltpu.make_async_copy(
            hp_sh.at[srcix_v.at[0]], rows_v.at[0], gsem).wait()

        @pl.when(g + 1 < NG)
        def _():
            wait_idx()
            pltpu.async_copy(
                hp_sh.at[srcix_v.at[(g + 1) % 2]], rows_v.at[(g + 1) % 2],
                gsem)

        pltpu.sync_copy(rows_v.at[g % 2], acc_sh.at[dstix_v.at[g % 2]],
                        add=True)

        @pl.when(g + 2 < NG)
        def _():
            load_idx(g + 2, g % 2)

        return carry

    lax.fori_loop(0, NG, chunk, 0)
    plsc.subcore_barrier()
    pltpu.sync_copy(acc_sh.at[pl.ds(s * RPT, RPT)],
                    out_hbm.at[c].at[pl.ds(s * RPT, RPT)])


_prop_call = pl.kernel(
    _prop_body,
    out_type=jax.ShapeDtypeStruct((NC, TBL_ROWS, H), jnp.float32),
    mesh=_MESH,
    scratch_types=[
        pltpu.VMEM((2, GC), jnp.int32),
        pltpu.VMEM((2, GC), jnp.int32),
        pltpu.VMEM((2, GC, H), jnp.float32),
        pltpu.SemaphoreType.DMA,
        pltpu.SemaphoreType.DMA,
        pltpu.VMEM_SHARED((TBL_ROWS, H), jnp.float32),
        pltpu.VMEM_SHARED((TBL_ROWS, H), jnp.float32),
    ],
    compiler_params=_SC_PARAMS,
)


def _deg_body(ei_hbm, ones_hbm, zeros_hbm, out_hbm,
              dstix_v, ones_v, isem, acc_sh):
    c = lax.axis_index("c")
    s = lax.axis_index("s")
    base = (c * NS + s) * EPT
    pltpu.sync_copy(zeros_hbm, acc_sh.at[pl.ds(s * RPT, RPT)])
    pltpu.sync_copy(ones_hbm, ones_v)
    pltpu.async_copy(ei_hbm.at[1].at[pl.ds(base, GC)], dstix_v.at[0], isem)
    plsc.subcore_barrier()

    def chunk(g, carry):
        pltpu.make_async_copy(
            ei_hbm.at[1].at[pl.ds(0, GC)], dstix_v.at[0], isem).wait()

        @pl.when(g + 1 < NG)
        def _():
            pltpu.async_copy(
                ei_hbm.at[1].at[pl.ds(base + (g + 1) * GC, GC)],
                dstix_v.at[(g + 1) % 2], isem)

        pltpu.sync_copy(ones_v, acc_sh.at[dstix_v.at[g % 2]], add=True)
        return carry

    lax.fori_loop(0, NG, chunk, 0)
    plsc.subcore_barrier()
    pltpu.sync_copy(acc_sh.at[pl.ds(s * RPT, RPT)],
                    out_hbm.at[c].at[pl.ds(s * RPT, RPT)])


_deg_call = pl.kernel(
    _deg_body,
    out_type=jax.ShapeDtypeStruct((NC, TBL_ROWS, H), jnp.float32),
    mesh=_MESH,
    scratch_types=[
        pltpu.VMEM((2, GC), jnp.int32),
        pltpu.VMEM((GC, H), jnp.float32),
        pltpu.SemaphoreType.DMA,
        pltpu.VMEM_SHARED((TBL_ROWS, H), jnp.float32),
    ],
    compiler_params=_SC_PARAMS,
)


def _dotf(a, b):
    return jnp.dot(a, b, preferred_element_type=jnp.float32)


def _dotbf(a, b):
    return jnp.dot(a.astype(jnp.bfloat16), b.astype(jnp.bfloat16),
                   preferred_element_type=jnp.float32)


def _sel(rows, cols, mod):
    i = lax.broadcasted_iota(jnp.int32, (rows, cols), 0)
    j = lax.broadcasted_iota(jnp.int32, (rows, cols), 1)
    return jnp.where(i % mod == j % mod, 1.0, 0.0).astype(jnp.float32)


def _blockmask(rows, cols, bi, bj):
    i = lax.broadcasted_iota(jnp.int32, (rows, cols), 0)
    j = lax.broadcasted_iota(jnp.int32, (rows, cols), 1)
    return jnp.where(i // bi == j // bj, 1.0, 0.0).astype(jnp.float32)


def _blockdiag(w, copies):
    bi, bj = w.shape
    a = _sel(copies * bi, bi, bi)
    b = _sel(bj, copies * bj, bj)
    return _dotf(_dotf(a, w), b) * _blockmask(copies * bi, copies * bj,
                                              bi, bj)


def _tile_lanes(v, copies):
    return jnp.concatenate([v[None, :]] * copies, axis=1)


def _group_reduce(s, k):
    copies = s.shape[1] // k
    tot = s[:, 0:k]
    for a in range(1, copies):
        tot = tot + s[:, a * k:(a + 1) * k]
    return _tile_lanes(tot[0], copies)


def _colsum(p):
    return _dotf(jnp.ones((1, p.shape[0]), jnp.float32), p)


def _bn_gelu_packed(p, g_t, be_t, k):
    s1 = _group_reduce(_colsum(p), k)
    m = s1 * (1.0 / N)
    d = p - m
    s2 = _group_reduce(_colsum(d * d), k)
    var = s2 * (1.0 / N)
    return jax.nn.gelu(d / jnp.sqrt(var + 1e-5) * g_t + be_t)


def _packed_matmul(u_p, w, dot=None):
    dot = dot or _dotbf
    k = w.shape[0]
    return jnp.concatenate(
        [dot(u_p[:, a * k:(a + 1) * k], w) for a in range(8)], axis=1)


def _tc_pre_body(degacc_ref, xp_ref, w1_ref, dinv_ref, hp1_ref):
    dinv_p = lax.rsqrt(degacc_ref[0, :PN, :] + degacc_ref[1, :PN, :] + 1.0)
    t1_p = _packed_matmul(xp_ref[...], w1_ref[...])
    dinv_ref[:PN, :] = dinv_p
    hp1_ref[:PN, :] = t1_p * dinv_p


def _tc_mid_body(acc_ref, hp_ref, dinv_ref, b_ref, g_ref, be_ref, wn_ref,
                 hpn_ref):
    dinv_p = dinv_ref[:PN, :]
    p_p = (acc_ref[0, :PN, :] + acc_ref[1, :PN, :] + hp_ref[:PN, :]) * dinv_p
    u = _bn_gelu_packed(p_p + _tile_lanes(b_ref[...], 8),
                        _tile_lanes(g_ref[...], 8),
                        _tile_lanes(be_ref[...], 8), H)
    hpn_ref[:PN, :] = _packed_matmul(u, wn_ref[...]) * dinv_p


def _tc_mid2_body(acc_ref, hp_ref, dinv_ref, b_ref, g_ref, be_ref, hpn_ref):
    dinv_p = dinv_ref[:PN, :]
    p_p = (acc_ref[0, :PN, :] + acc_ref[1, :PN, :] + hp_ref[:PN, :]) * dinv_p
    u = _bn_gelu_packed(p_p + _tile_lanes(b_ref[...], 8),
                        _tile_lanes(g_ref[...], 8),
                        _tile_lanes(be_ref[...], 8), H)
    hpn_ref[:PN, :] = u * dinv_p


def _tc_post_body(acc_ref, hp_ref, dinv_ref, x_ref, w3_ref, b3_ref, g3_ref,
                  be3_ref, fw1_ref, fb1_ref, fw2_ref, fb2_ref, fw3_ref,
                  fb3_ref, fw4_ref, fb4_ref, out_ref):
    p3_p = (acc_ref[0, :PN, :] + acc_ref[1, :PN, :] + hp_ref[:PN, :]) \
        * dinv_ref[:PN, :]
    t3_p = _packed_matmul(p3_p, w3_ref[...], dot=_dotf)
    u3_p = _bn_gelu_packed(t3_p + _tile_lanes(b3_ref[...], 8),
                           _tile_lanes(g3_ref[...], 8),
                           _tile_lanes(be3_ref[...], 8), D)
    vu = _group_reduce(_colsum(u3_p), D)[:, 0:D]
    v = (_colsum(x_ref[...]) + vu) * (1.0 / N)
    v = jax.nn.gelu(_dotbf(v, fw1_ref[...]) + fb1_ref[...])
    v = jax.nn.gelu(_dotbf(v, fw2_ref[...]) + fb2_ref[...])
    v = jax.nn.gelu(_dotbf(v, fw3_ref[...]) + fb3_ref[...])
    out_ref[...] = _dotbf(v, fw4_ref[...]) + fb4_ref[...]


_pre_call = pl.pallas_call(_tc_pre_body, out_shape=(
    jax.ShapeDtypeStruct((PR, 128), jnp.float32),
    jax.ShapeDtypeStruct((PR, 128), jnp.float32),
))
_mid_call = pl.pallas_call(
    _tc_mid_body, out_shape=jax.ShapeDtypeStruct((PR, 128), jnp.float32))
_mid2_call = pl.pallas_call(
    _tc_mid2_body, out_shape=jax.ShapeDtypeStruct((PR, 128), jnp.float32))
_post_call = pl.pallas_call(
    _tc_post_body, out_shape=jax.ShapeDtypeStruct((1, 1), jnp.float32))


def kernel(x, edge_index, W1, b1, g1, be1, W2, b2, g2, be2, W3, b3, g3, be3,
           fw1, fb1, fw2, fb2, fw3, fb3, fw4, fb4):
    ei = edge_index.astype(jnp.int32)
    zeros_rows = jnp.zeros((RPT, H), jnp.float32)
    ones_rows = jnp.ones((GC, H), jnp.float32)

    def pack(a):
        return jnp.reshape(a, a.shape[:-2] + (PR, 128))

    def unpack(a):
        return jnp.reshape(a, (TBL_ROWS, H))

    degacc = _deg_call(ei, ones_rows, zeros_rows)
    dinv, hp1 = _pre_call(pack(degacc), x.reshape(PN, 8 * D), W1)

    acc1 = _prop_call(unpack(hp1), ei, zeros_rows)
    hp2 = _mid_call(pack(acc1), hp1, dinv, b1, g1, be1, W2)

    acc2 = _prop_call(unpack(hp2), ei, zeros_rows)
    hp3 = _mid2_call(pack(acc2), hp2, dinv, b2, g2, be2)

    acc3 = _prop_call(unpack(hp3), ei, zeros_rows)
    out = _post_call(pack(acc3), hp3, dinv, x, W3, b3, g3, be3,
                     fw1, fb1, fw2, fb2, fw3, fb3, fw4, fb4)
    return out.reshape(1)
